# baseline (speedup 1.0000x reference)
import jax
import jax.numpy as jnp
from jax import lax
from jax.experimental import pallas as pl
from jax.experimental.pallas import tpu as pltpu


_MIB = 1024 * 1024
_VMEM_LIMIT = 52 * _MIB


def _ff_relu_kernel(x_ref, w1_ref, b1_ref, h_ref):
    x = x_ref[...].astype(jnp.bfloat16)
    w = w1_ref[...].astype(jnp.bfloat16)
    h = lax.dot_general(
        x, w,
        dimension_numbers=(((1,), (1,)), ((), ())),
        preferred_element_type=jnp.float32)
    h = h + b1_ref[...]
    h_ref[...] = jnp.maximum(h, 0.0).astype(h_ref.dtype)


def _vocab_proj_kernel(h_ref, emb_ref, o_ref):
    e = emb_ref[...].astype(jnp.bfloat16)
    logits = lax.dot_general(
        h_ref[...], e,
        dimension_numbers=(((1,), (1,)), ((), ())),
        preferred_element_type=jnp.float32)
    o_ref[...] = logits.astype(o_ref.dtype)


def _pick_vocab_tile(V):
    for tv in (1280, 1024, 640, 512, 256, 128):
        if V % tv == 0 and (V // tv) % 2 == 0:
            return tv
    return min(V, 512)


def kernel(x, w1, b1, emb):
    B, S, D = x.shape
    V, D_e = emb.shape
    assert D_e == D
    M = B * S

    xm = x.reshape(M, D)
    b1_2d = b1.reshape(1, D)

    tm = M // 2 if M % 2 == 0 else M
    ff_grid = (M // tm,)
    ff_cost = pl.CostEstimate(
        flops=2 * M * D * D,
        transcendentals=0,
        bytes_accessed=M * D * 4 + (M // tm) * D * D * 4 + M * D * 2)

    h = pl.pallas_call(
        _ff_relu_kernel,
        out_shape=jax.ShapeDtypeStruct((M, D), jnp.bfloat16),
        grid=ff_grid,
        in_specs=[
            pl.BlockSpec((tm, D), lambda i: (i, 0)),
            pl.BlockSpec((D, D), lambda i: (0, 0)),
            pl.BlockSpec((1, D), lambda i: (0, 0)),
        ],
        out_specs=pl.BlockSpec((tm, D), lambda i: (i, 0)),
        compiler_params=pltpu.CompilerParams(
            dimension_semantics=("parallel",),
            vmem_limit_bytes=_VMEM_LIMIT),
        cost_estimate=ff_cost,
    )(xm, w1, b1_2d)

    tv = _pick_vocab_tile(V)
    grid = (pl.cdiv(V, tv),)
    cost = pl.CostEstimate(
        flops=2 * M * D * V,
        transcendentals=0,
        bytes_accessed=M * D * 2 + V * D * 4 + M * V * 4)

    out = pl.pallas_call(
        _vocab_proj_kernel,
        out_shape=jax.ShapeDtypeStruct((M, V), x.dtype),
        grid=grid,
        in_specs=[
            pl.BlockSpec((M, D), lambda j: (0, 0)),
            pl.BlockSpec((tv, D), lambda j: (j, 0)),
        ],
        out_specs=pl.BlockSpec((M, tv), lambda j: (0, j)),
        compiler_params=pltpu.CompilerParams(
            dimension_semantics=("parallel",),
            vmem_limit_bytes=_VMEM_LIMIT),
        cost_estimate=cost,
    )(h, emb)

    return out.reshape(B, S, V)


# speedup vs baseline: 1.2765x; 1.2765x over previous
import jax
import jax.numpy as jnp
from jax import lax
from jax.experimental import pallas as pl
from jax.experimental.pallas import tpu as pltpu


_MIB = 1024 * 1024
_VMEM_LIMIT = 52 * _MIB


def _fused_kernel(x_ref, w1_ref, b1_ref, emb_ref, o_ref, h_ref):
    @pl.when(pl.program_id(0) == 0)
    def _compute_h():
        h = lax.dot_general(
            x_ref[...], w1_ref[...],
            dimension_numbers=(((1,), (1,)), ((), ())),
            preferred_element_type=jnp.float32)
        h_ref[...] = jnp.maximum(h + b1_ref[...], 0.0)

    o_ref[...] = lax.dot_general(
        h_ref[...], emb_ref[...],
        dimension_numbers=(((1,), (1,)), ((), ())),
        preferred_element_type=jnp.float32).astype(o_ref.dtype)


def _pick_vocab_tile(V):
    for tv in (2048, 1280, 1024, 640, 512, 256, 128):
        if V % tv == 0:
            return tv
    return min(V, 512)


def kernel(x, w1, b1, emb):
    B, S, D = x.shape
    V, D_e = emb.shape
    assert D_e == D
    M = B * S

    xm = x.reshape(M, D)
    b1_2d = b1.reshape(1, D)

    tv = _pick_vocab_tile(V)
    grid = (pl.cdiv(V, tv),)

    cost = pl.CostEstimate(
        flops=2 * M * D * (V + D),
        transcendentals=0,
        bytes_accessed=M * D * 4 + D * D * 4 + V * D * 4 + M * V * 4)

    out = pl.pallas_call(
        _fused_kernel,
        out_shape=jax.ShapeDtypeStruct((M, V), x.dtype),
        grid=grid,
        in_specs=[
            pl.BlockSpec((M, D), lambda j: (0, 0)),
            pl.BlockSpec((D, D), lambda j: (0, 0)),
            pl.BlockSpec((1, D), lambda j: (0, 0)),
            pl.BlockSpec((tv, D), lambda j: (j, 0)),
        ],
        out_specs=pl.BlockSpec((M, tv), lambda j: (0, j)),
        scratch_shapes=[pltpu.VMEM((M, D), jnp.float32)],
        compiler_params=pltpu.CompilerParams(
            dimension_semantics=("arbitrary",),
            vmem_limit_bytes=_VMEM_LIMIT),
        cost_estimate=cost,
    )(xm, w1, b1_2d, emb)

    return out.reshape(B, S, V)


# speedup vs baseline: 1.2795x; 1.0023x over previous
import jax
import jax.numpy as jnp
from jax import lax
from jax.experimental import pallas as pl
from jax.experimental.pallas import tpu as pltpu


_MIB = 1024 * 1024
_VMEM_LIMIT = 52 * _MIB


def _fused_kernel(x_ref, w1_ref, b1_ref, emb_ref, o_ref, h_ref):
    @pl.when(pl.program_id(0) == 0)
    def _compute_h():
        h = lax.dot_general(
            x_ref[...].astype(jnp.bfloat16), w1_ref[...].astype(jnp.bfloat16),
            dimension_numbers=(((1,), (1,)), ((), ())),
            preferred_element_type=jnp.float32)
        h_ref[...] = jnp.maximum(h + b1_ref[...], 0.0).astype(h_ref.dtype)

    o_ref[...] = lax.dot_general(
        h_ref[...], emb_ref[...].astype(jnp.bfloat16),
        dimension_numbers=(((1,), (1,)), ((), ())),
        preferred_element_type=jnp.float32).astype(o_ref.dtype)


def _pick_vocab_tile(V):
    for tv in (2048, 1280, 1024, 640, 512, 256, 128):
        if V % tv == 0:
            return tv
    return min(V, 512)


def kernel(x, w1, b1, emb):
    B, S, D = x.shape
    V, D_e = emb.shape
    assert D_e == D
    M = B * S

    xm = x.reshape(M, D)
    b1_2d = b1.reshape(1, D)

    tv = _pick_vocab_tile(V)
    grid = (pl.cdiv(V, tv),)

    cost = pl.CostEstimate(
        flops=2 * M * D * (V + D),
        transcendentals=0,
        bytes_accessed=M * D * 4 + D * D * 4 + V * D * 4 + M * V * 4)

    out = pl.pallas_call(
        _fused_kernel,
        out_shape=jax.ShapeDtypeStruct((M, V), x.dtype),
        grid=grid,
        in_specs=[
            pl.BlockSpec((M, D), lambda j: (0, 0)),
            pl.BlockSpec((D, D), lambda j: (0, 0)),
            pl.BlockSpec((1, D), lambda j: (0, 0)),
            pl.BlockSpec((tv, D), lambda j: (j, 0)),
        ],
        out_specs=pl.BlockSpec((M, tv), lambda j: (0, j)),
        scratch_shapes=[pltpu.VMEM((M, D), jnp.bfloat16)],
        compiler_params=pltpu.CompilerParams(
            dimension_semantics=("arbitrary",),
            vmem_limit_bytes=_VMEM_LIMIT),
        cost_estimate=cost,
    )(xm, w1, b1_2d, emb)

    return out.reshape(B, S, V)


# speedup vs baseline: 1.3320x; 1.0410x over previous
import jax
import jax.numpy as jnp
from jax import lax
from jax.experimental import pallas as pl
from jax.experimental.pallas import tpu as pltpu


_MIB = 1024 * 1024
_VMEM_LIMIT = 52 * _MIB


def _fused_kernel(x_ref, w1_ref, b1_ref, emb_ref, o_ref, h_ref):
    @pl.when(pl.program_id(0) == 0)
    def _compute_h():
        h = lax.dot_general(
            x_ref[...].astype(jnp.bfloat16), w1_ref[...].astype(jnp.bfloat16),
            dimension_numbers=(((1,), (1,)), ((), ())),
            preferred_element_type=jnp.float32)
        h_ref[...] = jnp.maximum(h + b1_ref[...], 0.0).astype(h_ref.dtype)

    o_ref[...] = lax.dot_general(
        h_ref[...], emb_ref[...].astype(jnp.bfloat16),
        dimension_numbers=(((1,), (1,)), ((), ())),
        preferred_element_type=jnp.float32).astype(o_ref.dtype)


def _pick_vocab_tile(V):
    for tv in (3200, 2048, 1280, 1024, 640, 512, 256, 128):
        if V % tv == 0:
            return tv
    return min(V, 512)


def kernel(x, w1, b1, emb):
    B, S, D = x.shape
    V, D_e = emb.shape
    assert D_e == D
    M = B * S

    xm = x.reshape(M, D)
    b1_2d = b1.reshape(1, D)

    tv = _pick_vocab_tile(V)
    grid = (pl.cdiv(V, tv),)

    cost = pl.CostEstimate(
        flops=2 * M * D * (V + D),
        transcendentals=0,
        bytes_accessed=M * D * 4 + D * D * 4 + V * D * 4 + M * V * 4)

    out = pl.pallas_call(
        _fused_kernel,
        out_shape=jax.ShapeDtypeStruct((M, V), x.dtype),
        grid=grid,
        in_specs=[
            pl.BlockSpec((M, D), lambda j: (0, 0)),
            pl.BlockSpec((D, D), lambda j: (0, 0)),
            pl.BlockSpec((1, D), lambda j: (0, 0)),
            pl.BlockSpec((tv, D), lambda j: (j, 0)),
        ],
        out_specs=pl.BlockSpec((M, tv), lambda j: (0, j)),
        scratch_shapes=[pltpu.VMEM((M, D), jnp.bfloat16)],
        compiler_params=pltpu.CompilerParams(
            dimension_semantics=("arbitrary",),
            vmem_limit_bytes=_VMEM_LIMIT),
        cost_estimate=cost,
    )(xm, w1, b1_2d, emb)

    return out.reshape(B, S, V)


# speedup vs baseline: 1.3561x; 1.0181x over previous
import jax
import jax.numpy as jnp
from jax import lax
from jax.experimental import pallas as pl
from jax.experimental.pallas import tpu as pltpu


_MIB = 1024 * 1024
_VMEM_LIMIT = 60 * _MIB


def _fused_kernel(x_ref, w1_ref, b1_ref, emb_ref, o_ref, h_ref):
    @pl.when(pl.program_id(0) == 0)
    def _compute_h():
        h = lax.dot_general(
            x_ref[...].astype(jnp.bfloat16), w1_ref[...].astype(jnp.bfloat16),
            dimension_numbers=(((1,), (1,)), ((), ())),
            preferred_element_type=jnp.float32)
        h_ref[...] = jnp.maximum(h + b1_ref[...], 0.0).astype(h_ref.dtype)

    o_ref[...] = lax.dot_general(
        h_ref[...], emb_ref[...].astype(jnp.bfloat16),
        dimension_numbers=(((1,), (1,)), ((), ())),
        preferred_element_type=jnp.float32).astype(o_ref.dtype)


def _pick_vocab_tile(V):
    for tv in (4096, 3200, 2048, 1280, 1024, 640, 512, 256, 128):
        if tv <= V:
            return tv
    return V


def kernel(x, w1, b1, emb):
    B, S, D = x.shape
    V, D_e = emb.shape
    assert D_e == D
    M = B * S

    xm = x.reshape(M, D)
    b1_2d = b1.reshape(1, D)

    tv = _pick_vocab_tile(V)
    grid = (pl.cdiv(V, tv),)

    cost = pl.CostEstimate(
        flops=2 * M * D * (V + D),
        transcendentals=0,
        bytes_accessed=M * D * 4 + D * D * 4 + V * D * 4 + M * V * 4)

    out = pl.pallas_call(
        _fused_kernel,
        out_shape=jax.ShapeDtypeStruct((M, V), x.dtype),
        grid=grid,
        in_specs=[
            pl.BlockSpec((M, D), lambda j: (0, 0)),
            pl.BlockSpec((D, D), lambda j: (0, 0)),
            pl.BlockSpec((1, D), lambda j: (0, 0)),
            pl.BlockSpec((tv, D), lambda j: (j, 0)),
        ],
        out_specs=pl.BlockSpec((M, tv), lambda j: (0, j)),
        scratch_shapes=[pltpu.VMEM((M, D), jnp.bfloat16)],
        compiler_params=pltpu.CompilerParams(
            dimension_semantics=("arbitrary",),
            vmem_limit_bytes=_VMEM_LIMIT),
        cost_estimate=cost,
    )(xm, w1, b1_2d, emb)

    return out.reshape(B, S, V)
